# baseline (speedup 1.0000x reference)
import functools

import jax
import jax.numpy as jnp
from jax import lax
from jax.experimental import pallas as pl
from jax.experimental.pallas import tpu as pltpu
from jax.experimental.pallas import tpu_sc as plsc

EMB_DIM = 32
HW = 1300
HWP = 1312
NCHUNK = 41
B = 1024
ROWS = B


def _embed_sc(idx_lin, emb_weight):
    info = plsc.get_sparse_core_info()
    nc, ns = info.num_cores, info.num_subcores
    nw = nc * ns
    assert nw * NCHUNK == HWP
    tile_r = (HW * EMB_DIM) // 8

    mesh = plsc.VectorSubcoreMesh(core_axis_name="c", subcore_axis_name="s")

    @functools.partial(
        pl.kernel,
        mesh=mesh,
        compiler_params=pltpu.CompilerParams(
            use_tc_tiling_on_sc=False, needs_layout_passes=False
        ),
        out_type=jax.ShapeDtypeStruct((tile_r, 64, 128), jnp.float32),
        scratch_types=[
            pltpu.VMEM((ROWS,), jnp.int32),
            pltpu.VMEM((ROWS,), jnp.int32),
            pltpu.VMEM((ROWS, EMB_DIM), jnp.float32),
            pltpu.VMEM((ROWS, EMB_DIM), jnp.float32),
            pltpu.VMEM((4, 64, 128), jnp.float32),
            pltpu.SemaphoreType.DMA,
            pltpu.SemaphoreType.DMA,
            pltpu.SemaphoreType.DMA,
            pltpu.SemaphoreType.DMA,
            pltpu.SemaphoreType.DMA,
        ],
    )
    def embed_kernel(
        idx_hbm,
        table_hbm,
        out_hbm,
        idx_b0,
        idx_b1,
        rows_b0,
        rows_b1,
        out_v,
        isem0,
        isem1,
        gsem0,
        gsem1,
        osem,
    ):
        idx_b = (idx_b0, idx_b1)
        rows_b = (rows_b0, rows_b1)
        isem = (isem0, isem1)
        gsem = (gsem0, gsem1)

        wid = lax.axis_index("s") * nc + lax.axis_index("c")
        base_i = wid * (NCHUNK * ROWS)
        hw0 = wid * NCHUNK
        iota16 = lax.iota(jnp.int32, 16)

        def start_idx(s, p):
            pltpu.async_copy(
                idx_hbm.at[pl.ds(base_i + s * ROWS, ROWS)], idx_b[p], isem[p]
            )

        def wait_idx(p):
            pltpu.make_async_copy(
                idx_hbm.at[pl.ds(0, ROWS)], idx_b[p], isem[p]
            ).wait()

        def start_gather(p):
            pltpu.async_copy(table_hbm.at[idx_b[p]], rows_b[p], gsem[p])

        def wait_gather(p):
            pltpu.make_async_copy(
                table_hbm.at[idx_b[p]], rows_b[p], gsem[p]
            ).wait()

        def out_dst(s):
            return out_hbm.at[pl.ds((hw0 + s) * 4, 4), :, :]

        def start_out(s):
            pltpu.async_copy(out_v, out_dst(s), osem)

        def wait_out(s):
            pltpu.make_async_copy(out_v, out_dst(s), osem).wait()

        def transpose_chunk(rb):
            @plsc.parallel_loop(0, ROWS // 16, unroll=8)
            def bh_body(bh):
                r_ids = bh * 16 + iota16
                v2 = (bh & 7) * 16 + iota16
                tb8 = (bh // 8) * 8
                for dh in range(2):
                    for k in range(16):
                        dvec = (iota16 + k) & 15
                        d_ids = dh * 16 + dvec
                        v = plsc.load_gather(rb, [r_ids, d_ids])
                        v0 = dh * 2 + (dvec >> 3)
                        v1 = tb8 + (dvec & 7)
                        plsc.store_scatter(out_v, [v0, v1, v2], v)

        valid_chunks = jnp.minimum(
            jnp.maximum(HW - hw0, 0), NCHUNK
        )

        start_idx(0, 0)
        wait_idx(0)
        start_gather(0)
        start_idx(1, 1)

        def loop_body(i, carry):
            for sl in range(2):
                s = 2 * i + sl
                p = sl
                wait_idx(1 - p)
                start_gather(1 - p)
                wait_gather(p)

                @pl.when(s + 2 < NCHUNK)
                def _():
                    start_idx(s + 2, p)

                @pl.when(s < valid_chunks)
                def _():
                    @pl.when(jnp.logical_and(s >= 1, s - 1 < valid_chunks))
                    def _():
                        wait_out(s - 1)

                    transpose_chunk(rows_b[p])
                    start_out(s)
            return carry

        lax.fori_loop(0, (NCHUNK - 1) // 2, loop_body, 0)

        s_last = NCHUNK - 1
        wait_gather(0)

        @pl.when(s_last < valid_chunks)
        def _():
            wait_out(s_last - 1)
            transpose_chunk(rows_b[0])
            start_out(s_last)
            wait_out(s_last)

        @pl.when(
            jnp.logical_and(s_last >= valid_chunks, valid_chunks >= 1)
        )
        def _():
            wait_out(valid_chunks - 1)

    return embed_kernel(idx_lin, emb_weight)


def kernel(inputs, emb_weight):
    b, h, w = inputs.shape
    assert h * w == HW and b == B
    idx = inputs.reshape(b, HW).astype(jnp.int32)
    idx_pad = jnp.pad(idx, ((0, 0), (0, HWP - HW)))
    idx_lin = idx_pad.T.reshape(-1)
    out3 = _embed_sc(idx_lin, emb_weight)
    x = out3.reshape(HW, 4, 8, 8, 128)
    x = x.transpose(2, 4, 1, 3, 0)
    return x.reshape(b, EMB_DIM, h, w)


# speedup vs baseline: 1.1100x; 1.1100x over previous
import functools

import jax
import jax.numpy as jnp
from jax import lax
from jax.experimental import pallas as pl
from jax.experimental.pallas import tpu as pltpu
from jax.experimental.pallas import tpu_sc as plsc

EMB_DIM = 32
HW = 1300
HWP = 1312
NCHUNK = 41
B = 1024
ROWS = B


def _embed_sc(idx_lin, emb_weight):
    info = plsc.get_sparse_core_info()
    nc, ns = info.num_cores, info.num_subcores
    nw = nc * ns
    assert nw * NCHUNK == HWP
    tile_r = (HW * EMB_DIM) // 8

    mesh = plsc.VectorSubcoreMesh(core_axis_name="c", subcore_axis_name="s")

    @functools.partial(
        pl.kernel,
        mesh=mesh,
        compiler_params=pltpu.CompilerParams(
            use_tc_tiling_on_sc=False, needs_layout_passes=False
        ),
        out_type=jax.ShapeDtypeStruct((tile_r, 64, 128), jnp.float32),
        scratch_types=[
            pltpu.VMEM((ROWS,), jnp.int32),
            pltpu.VMEM((ROWS,), jnp.int32),
            pltpu.VMEM((ROWS, EMB_DIM), jnp.float32),
            pltpu.VMEM((ROWS, EMB_DIM), jnp.float32),
            pltpu.VMEM((4, 64, 128), jnp.float32),
            pltpu.SemaphoreType.DMA,
            pltpu.SemaphoreType.DMA,
            pltpu.SemaphoreType.DMA,
            pltpu.SemaphoreType.DMA,
            pltpu.SemaphoreType.DMA,
        ],
    )
    def embed_kernel(
        idx_hbm,
        table_hbm,
        out_hbm,
        idx_b0,
        idx_b1,
        rows_b0,
        rows_b1,
        out_v,
        isem0,
        isem1,
        gsem0,
        gsem1,
        osem,
    ):
        idx_b = (idx_b0, idx_b1)
        rows_b = (rows_b0, rows_b1)
        isem = (isem0, isem1)
        gsem = (gsem0, gsem1)

        wid = lax.axis_index("s") * nc + lax.axis_index("c")
        base_i = wid * (NCHUNK * ROWS)
        hw0 = wid * NCHUNK
        iota16 = lax.iota(jnp.int32, 16)

        def start_idx(s, p):
            pltpu.async_copy(
                idx_hbm.at[pl.ds(base_i + s * ROWS, ROWS)], idx_b[p], isem[p]
            )

        def wait_idx(p):
            pltpu.make_async_copy(
                idx_hbm.at[pl.ds(0, ROWS)], idx_b[p], isem[p]
            ).wait()

        def start_gather(p):
            pltpu.async_copy(table_hbm.at[idx_b[p]], rows_b[p], gsem[p])

        def wait_gather(p):
            pltpu.make_async_copy(
                table_hbm.at[idx_b[p]], rows_b[p], gsem[p]
            ).wait()

        def out_dst(s):
            return out_hbm.at[pl.ds((hw0 + s) * 4, 4), :, :]

        def start_out(s):
            pltpu.async_copy(out_v, out_dst(s), osem)

        def wait_out(s):
            pltpu.make_async_copy(out_v, out_dst(s), osem).wait()

        def transpose_chunk(rb):
            @plsc.parallel_loop(0, ROWS // 16, unroll=4)
            def bh_body(bh):
                r_ids = bh * 16 + iota16
                v2 = (bh & 7) * 16 + iota16
                tb8 = (bh // 8) * 8
                for dh in range(2):
                    for k in range(16):
                        dvec = (iota16 + k) & 15
                        d_ids = dh * 16 + dvec
                        v = plsc.load_gather(rb, [r_ids, d_ids])
                        v0 = dh * 2 + (dvec >> 3)
                        v1 = tb8 + (dvec & 7)
                        plsc.store_scatter(out_v, [v0, v1, v2], v)

        valid_chunks = jnp.minimum(
            jnp.maximum(HW - hw0, 0), NCHUNK
        )

        start_idx(0, 0)
        wait_idx(0)
        start_gather(0)
        start_idx(1, 1)

        def loop_body(i, carry):
            for sl in range(2):
                s = 2 * i + sl
                p = sl
                wait_idx(1 - p)
                start_gather(1 - p)
                wait_gather(p)

                @pl.when(s + 2 < NCHUNK)
                def _():
                    start_idx(s + 2, p)

                @pl.when(s < valid_chunks)
                def _():
                    @pl.when(jnp.logical_and(s >= 1, s - 1 < valid_chunks))
                    def _():
                        wait_out(s - 1)

                    transpose_chunk(rows_b[p])
                    start_out(s)
            return carry

        lax.fori_loop(0, (NCHUNK - 1) // 2, loop_body, 0)

        s_last = NCHUNK - 1
        wait_gather(0)

        @pl.when(s_last < valid_chunks)
        def _():
            wait_out(s_last - 1)
            transpose_chunk(rows_b[0])
            start_out(s_last)
            wait_out(s_last)

        @pl.when(
            jnp.logical_and(s_last >= valid_chunks, valid_chunks >= 1)
        )
        def _():
            wait_out(valid_chunks - 1)

    return embed_kernel(idx_lin, emb_weight)


def kernel(inputs, emb_weight):
    b, h, w = inputs.shape
    assert h * w == HW and b == B
    idx = inputs.reshape(b, HW).astype(jnp.int32)
    idx_pad = jnp.pad(idx, ((0, 0), (0, HWP - HW)))
    idx_lin = idx_pad.T.reshape(-1)
    out3 = _embed_sc(idx_lin, emb_weight)
    x = out3.reshape(HW, 4, 8, 8, 128)
    x = x.transpose(2, 4, 1, 3, 0)
    return x.reshape(b, EMB_DIM, h, w)


# speedup vs baseline: 1.1678x; 1.0521x over previous
import functools

import jax
import jax.numpy as jnp
from jax import lax
from jax.experimental import pallas as pl
from jax.experimental.pallas import tpu as pltpu
from jax.experimental.pallas import tpu_sc as plsc

EMB_DIM = 32
HW = 1300
HWP = 1312
NCHUNK = 41
B = 1024
ROWS = B


def _embed_sc(idx_lin, emb_weight):
    info = plsc.get_sparse_core_info()
    nc, ns = info.num_cores, info.num_subcores
    nw = nc * ns
    assert nw * NCHUNK == HWP
    tile_r = (HW * EMB_DIM) // 8

    mesh = plsc.VectorSubcoreMesh(core_axis_name="c", subcore_axis_name="s")

    @functools.partial(
        pl.kernel,
        mesh=mesh,
        compiler_params=pltpu.CompilerParams(
            use_tc_tiling_on_sc=False, needs_layout_passes=False
        ),
        out_type=jax.ShapeDtypeStruct((tile_r, 64, 128), jnp.float32),
        scratch_types=[
            pltpu.VMEM((ROWS,), jnp.int32),
            pltpu.VMEM((ROWS,), jnp.int32),
            pltpu.VMEM((ROWS, EMB_DIM), jnp.float32),
            pltpu.VMEM((ROWS, EMB_DIM), jnp.float32),
            pltpu.VMEM((2, 64, 128), jnp.float32),
            pltpu.VMEM((2, 64, 128), jnp.float32),
            pltpu.SemaphoreType.DMA,
            pltpu.SemaphoreType.DMA,
            pltpu.SemaphoreType.DMA,
            pltpu.SemaphoreType.DMA,
            pltpu.SemaphoreType.DMA,
            pltpu.SemaphoreType.DMA,
        ],
    )
    def embed_kernel(
        idx_hbm,
        table_hbm,
        out_hbm,
        idx_b0,
        idx_b1,
        rows_b0,
        rows_b1,
        out_v0,
        out_v1,
        isem0,
        isem1,
        gsem0,
        gsem1,
        osem0,
        osem1,
    ):
        idx_b = (idx_b0, idx_b1)
        rows_b = (rows_b0, rows_b1)
        out_v = (out_v0, out_v1)
        isem = (isem0, isem1)
        gsem = (gsem0, gsem1)
        osem = (osem0, osem1)

        wid = lax.axis_index("s") * nc + lax.axis_index("c")
        base_i = wid * (NCHUNK * ROWS)
        hw0 = wid * NCHUNK
        iota16 = lax.iota(jnp.int32, 16)

        def start_idx(s, p):
            pltpu.async_copy(
                idx_hbm.at[pl.ds(base_i + s * ROWS, ROWS)], idx_b[p], isem[p]
            )

        def wait_idx(p):
            pltpu.make_async_copy(
                idx_hbm.at[pl.ds(0, ROWS)], idx_b[p], isem[p]
            ).wait()

        def start_gather(p):
            pltpu.async_copy(table_hbm.at[idx_b[p]], rows_b[p], gsem[p])

        def wait_gather(p):
            pltpu.make_async_copy(
                table_hbm.at[idx_b[p]], rows_b[p], gsem[p]
            ).wait()

        def out_dst(s, dh):
            return out_hbm.at[pl.ds((hw0 + s) * 4 + dh * 2, 2), :, :]

        def start_out(s, dh):
            pltpu.async_copy(out_v[dh], out_dst(s, dh), osem[dh])

        def wait_out(s, dh):
            pltpu.make_async_copy(out_v[dh], out_dst(s, dh), osem[dh]).wait()

        def transpose_half(rb, dh):
            ov = out_v[dh]

            @plsc.parallel_loop(0, ROWS // 16, unroll=4)
            def bh_body(bh):
                r_ids = bh * 16 + iota16
                v2 = (bh & 7) * 16 + iota16
                tb8 = (bh // 8) * 8
                for k in range(16):
                    dvec = (iota16 + k) & 15
                    d_ids = dh * 16 + dvec
                    v = plsc.load_gather(rb, [r_ids, d_ids])
                    v0 = dvec >> 3
                    v1 = tb8 + (dvec & 7)
                    plsc.store_scatter(ov, [v0, v1, v2], v)

        valid_chunks = jnp.minimum(
            jnp.maximum(HW - hw0, 0), NCHUNK
        )

        start_idx(0, 0)
        wait_idx(0)
        start_gather(0)
        start_idx(1, 1)

        def loop_body(i, carry):
            for sl in range(2):
                s = 2 * i + sl
                p = sl
                wait_idx(1 - p)
                start_gather(1 - p)
                wait_gather(p)

                @pl.when(s + 2 < NCHUNK)
                def _():
                    start_idx(s + 2, p)

                @pl.when(s < valid_chunks)
                def _():
                    for dh in range(2):
                        @pl.when(s >= 1)
                        def _():
                            wait_out(s - 1, dh)

                        transpose_half(rows_b[p], dh)
                        start_out(s, dh)
            return carry

        lax.fori_loop(0, (NCHUNK - 1) // 2, loop_body, 0)

        s_last = NCHUNK - 1
        wait_gather(0)

        @pl.when(s_last < valid_chunks)
        def _():
            for dh in range(2):
                wait_out(s_last - 1, dh)
                transpose_half(rows_b[0], dh)
                start_out(s_last, dh)
            for dh in range(2):
                wait_out(s_last, dh)

        @pl.when(
            jnp.logical_and(s_last >= valid_chunks, valid_chunks >= 1)
        )
        def _():
            for dh in range(2):
                wait_out(valid_chunks - 1, dh)

    return embed_kernel(idx_lin, emb_weight)


def kernel(inputs, emb_weight):
    b, h, w = inputs.shape
    assert h * w == HW and b == B
    idx = inputs.reshape(b, HW).astype(jnp.int32)
    idx_pad = jnp.pad(idx, ((0, 0), (0, HWP - HW)))
    idx_lin = idx_pad.T.reshape(-1)
    out3 = _embed_sc(idx_lin, emb_weight)
    x = out3.reshape(HW, 4, 8, 8, 128)
    x = x.transpose(2, 4, 1, 3, 0)
    return x.reshape(b, EMB_DIM, h, w)
